# baseline (speedup 1.0000x reference)
import functools

import jax
import jax.numpy as jnp
from jax import lax
from jax.experimental import pallas as pl
from jax.experimental.pallas import tpu as pltpu
from jax.experimental.pallas import tpu_sc as plsc

_NC = 2
_NS = 16
_NW = _NC * _NS
_CH = 128


def _mlp_body(x_ref, ev_ref, w0_ref, b0_ref, w1_ref, b1_ref, w2_ref, b2_ref,
              w3_ref, b3_ref, w4_ref, b4_ref, out_ref):
    h = x_ref[...]
    for w_ref, b_ref in ((w0_ref, b0_ref), (w1_ref, b1_ref),
                         (w2_ref, b2_ref), (w3_ref, b3_ref)):
        g = jnp.dot(h.astype(jnp.bfloat16), w_ref[...],
                    preferred_element_type=jnp.float32)
        g = g + b_ref[...]
        h = g + g * jnp.tanh(g)
    s = jnp.dot(h, w4_ref[...], preferred_element_type=jnp.float32)
    s = s + b4_ref[...]
    out_ref[...] = s * jnp.pad(ev_ref[...], ((0, 0), (0, 5)))


def _mlp_forces(ff, ev, ws, bs, block_e, grid):
    h = ff.shape[1]
    row = lambda i: (i, 0)
    const = lambda i: (0, 0)
    in_specs = [
        pl.BlockSpec((block_e, h), row),
        pl.BlockSpec((block_e, 3), row),
    ]
    for w in ws[:4]:
        in_specs.append(pl.BlockSpec(w.shape, const))
        in_specs.append(pl.BlockSpec((1, h), const))
    in_specs.append(pl.BlockSpec((h, 8), const))
    in_specs.append(pl.BlockSpec((1, 8), const))
    args = [ff, ev]
    for w, b in zip(ws, bs):
        args.append(w)
        args.append(b)
    return pl.pallas_call(
        _mlp_body,
        grid=(grid,),
        in_specs=in_specs,
        out_specs=pl.BlockSpec((block_e, 8), row),
        out_shape=jax.ShapeDtypeStruct((ff.shape[0], 8), jnp.float32),
    )(*args)


def _scatter_partials(vals, idx1d, zeros):
    e = vals.shape[0]
    n_acc = zeros.shape[0]
    nchunks = e // _CH
    cpw = nchunks // _NW
    rem = nchunks - cpw * _NW
    epw = cpw * _CH
    rps = n_acc // _NS

    mesh = plsc.VectorSubcoreMesh(
        core_axis_name="c", subcore_axis_name="s",
        num_cores=_NC, num_subcores=_NS)

    @functools.partial(
        pl.kernel,
        out_type=jax.ShapeDtypeStruct((_NC, n_acc, 8), jnp.float32),
        mesh=mesh,
        scratch_types=[
            pltpu.VMEM((epw,), jnp.int32),
            pltpu.VMEM((epw, 8), jnp.float32),
            pltpu.VMEM((rem * _CH,), jnp.int32),
            pltpu.VMEM((rem * _CH, 8), jnp.float32),
            pltpu.VMEM_SHARED((n_acc, 8), jnp.float32),
            pltpu.SemaphoreType.DMA,
        ],
        compiler_params=pltpu.CompilerParams(use_tc_tiling_on_sc=False),
    )
    def scatter_kernel(vals_hbm, idx_hbm, zeros_hbm, out_hbm,
                       idx_v, vals_v, idx_x, vals_x, acc_sh, sem):
        cid = lax.axis_index("c")
        sid = lax.axis_index("s")
        wid = sid * _NC + cid
        rbase = sid * rps
        pltpu.sync_copy(zeros_hbm.at[pl.ds(rbase, rps)],
                        acc_sh.at[pl.ds(rbase, rps)])
        pltpu.sync_copy(idx_hbm.at[pl.ds(wid * epw, epw)], idx_v)
        pltpu.sync_copy(vals_hbm.at[pl.ds(wid * epw, epw)], vals_v)

        @pl.when(wid == 0)
        def _stage_tail():
            pltpu.sync_copy(idx_hbm.at[pl.ds(_NW * epw, rem * _CH)], idx_x)
            pltpu.sync_copy(vals_hbm.at[pl.ds(_NW * epw, rem * _CH)], vals_x)

        plsc.subcore_barrier()

        wave = 13
        for w in range(-(-cpw // wave)):
            cps = []
            for k in range(w * wave, min((w + 1) * wave, cpw)):
                cps.append(pltpu.async_copy(
                    vals_v.at[pl.ds(k * _CH, _CH)],
                    acc_sh.at[idx_v.at[pl.ds(k * _CH, _CH)]], sem, add=True))
            for cp in cps:
                cp.wait()

        @pl.when(wid == 0)
        def _scatter_tail():
            for k in range(rem):
                pltpu.sync_copy(vals_x.at[pl.ds(k * _CH, _CH)],
                                acc_sh.at[idx_x.at[pl.ds(k * _CH, _CH)]],
                                add=True)

        plsc.subcore_barrier()
        pltpu.sync_copy(acc_sh.at[pl.ds(rbase, rps)],
                        out_hbm.at[cid, pl.ds(rbase, rps)])

    return scatter_kernel(vals, idx1d, zeros)


def _combine_body(p_ref, o_ref):
    o_ref[...] = p_ref[0] + p_ref[1]


def _combine(partials):
    n_acc = partials.shape[1]
    return pl.pallas_call(
        _combine_body,
        out_shape=jax.ShapeDtypeStruct((n_acc, 8), jnp.float32),
    )(partials)


def kernel(force_features, edge_vectors, edge_index_dst, pos,
           W0, b0, W1, b1, W2, b2, W3, b3, W4, b4):
    e, h = force_features.shape
    n = pos.shape[0]

    block_e = 8192
    grid = -(-e // block_e)
    n_acc = -(-n // 128) * 128

    ws = [(0.5 * W0).astype(jnp.bfloat16), (0.5 * W1).astype(jnp.bfloat16),
          (0.5 * W2).astype(jnp.bfloat16), (0.5 * W3).astype(jnp.bfloat16),
          jnp.broadcast_to(W4, (h, 8))]
    bs = [0.5 * b0.reshape(1, h), 0.5 * b1.reshape(1, h),
          0.5 * b2.reshape(1, h), 0.5 * b3.reshape(1, h),
          jnp.broadcast_to(b4.reshape(1, 1), (1, 8))]
    zeros = jnp.zeros((n_acc, 8), jnp.float32)

    vals = _mlp_forces(force_features, edge_vectors, ws, bs, block_e, grid)
    partials = _scatter_partials(vals, edge_index_dst, zeros)
    forces_full = _combine(partials)
    return forces_full[:n, :3]


# speedup vs baseline: 1.1511x; 1.1511x over previous
import functools

import jax
import jax.numpy as jnp
from jax import lax
from jax.experimental import pallas as pl
from jax.experimental.pallas import tpu as pltpu
from jax.experimental.pallas import tpu_sc as plsc

_NC = 2
_NS = 16
_NW = _NC * _NS
_CH = 128


def _mlp_body(x_ref, ev_ref, w0_ref, b0_ref, w1_ref, b1_ref, w2_ref, b2_ref,
              w3_ref, b3_ref, w4_ref, b4_ref, out_ref):
    h = x_ref[...]
    for w_ref, b_ref in ((w0_ref, b0_ref), (w1_ref, b1_ref),
                         (w2_ref, b2_ref), (w3_ref, b3_ref)):
        g = jnp.dot(h.astype(jnp.bfloat16), w_ref[...],
                    preferred_element_type=jnp.float32)
        g = g + b_ref[...]
        h = g + g * jnp.tanh(g)
    s = jnp.dot(h, w4_ref[...], preferred_element_type=jnp.float32)
    s = s + b4_ref[...]
    ev = jnp.pad(ev_ref[...], ((0, 5), (0, 0))).T
    out_ref[...] = s * ev


def _mlp_forces(ff, ev, ws, bs, block_e, grid):
    h = ff.shape[1]
    row = lambda i: (i, 0)
    const = lambda i: (0, 0)
    in_specs = [
        pl.BlockSpec((block_e, h), row),
        pl.BlockSpec((3, block_e), lambda i: (0, i)),
    ]
    for w in ws[:4]:
        in_specs.append(pl.BlockSpec(w.shape, const))
        in_specs.append(pl.BlockSpec((1, h), const))
    in_specs.append(pl.BlockSpec((h, 8), const))
    in_specs.append(pl.BlockSpec((1, 8), const))
    args = [ff, ev]
    for w, b in zip(ws, bs):
        args.append(w)
        args.append(b)
    return pl.pallas_call(
        _mlp_body,
        grid=(grid,),
        in_specs=in_specs,
        out_specs=pl.BlockSpec((block_e, 8), row),
        out_shape=jax.ShapeDtypeStruct((ff.shape[0], 8), jnp.float32),
    )(*args)


def _scatter_partials(vals, idx1d, zeros):
    e = vals.shape[0]
    n_acc = zeros.shape[0]
    nchunks = e // _CH
    cpw = nchunks // _NW
    rem = nchunks - cpw * _NW
    epw = cpw * _CH
    rps = n_acc // _NS

    mesh = plsc.VectorSubcoreMesh(
        core_axis_name="c", subcore_axis_name="s",
        num_cores=_NC, num_subcores=_NS)

    @functools.partial(
        pl.kernel,
        out_type=jax.ShapeDtypeStruct((_NC, n_acc, 8), jnp.float32),
        mesh=mesh,
        scratch_types=[
            pltpu.VMEM((epw,), jnp.int32),
            pltpu.VMEM((epw, 8), jnp.float32),
            pltpu.VMEM((rem * _CH,), jnp.int32),
            pltpu.VMEM((rem * _CH, 8), jnp.float32),
            pltpu.VMEM_SHARED((n_acc, 8), jnp.float32),
            pltpu.SemaphoreType.DMA,
        ],
        compiler_params=pltpu.CompilerParams(use_tc_tiling_on_sc=False),
    )
    def scatter_kernel(vals_hbm, idx_hbm, zeros_hbm, out_hbm,
                       idx_v, vals_v, idx_x, vals_x, acc_sh, sem):
        cid = lax.axis_index("c")
        sid = lax.axis_index("s")
        wid = sid * _NC + cid
        rbase = sid * rps
        pltpu.sync_copy(zeros_hbm.at[pl.ds(rbase, rps)],
                        acc_sh.at[pl.ds(rbase, rps)])
        pltpu.sync_copy(idx_hbm.at[pl.ds(wid * epw, epw)], idx_v)
        pltpu.sync_copy(vals_hbm.at[pl.ds(wid * epw, epw)], vals_v)

        @pl.when(wid == 0)
        def _stage_tail():
            pltpu.sync_copy(idx_hbm.at[pl.ds(_NW * epw, rem * _CH)], idx_x)
            pltpu.sync_copy(vals_hbm.at[pl.ds(_NW * epw, rem * _CH)], vals_x)

        plsc.subcore_barrier()

        wave = 13
        for w in range(-(-cpw // wave)):
            cps = []
            for k in range(w * wave, min((w + 1) * wave, cpw)):
                cps.append(pltpu.async_copy(
                    vals_v.at[pl.ds(k * _CH, _CH)],
                    acc_sh.at[idx_v.at[pl.ds(k * _CH, _CH)]], sem, add=True))
            for cp in cps:
                cp.wait()

        @pl.when(wid == 0)
        def _scatter_tail():
            for k in range(rem):
                pltpu.sync_copy(vals_x.at[pl.ds(k * _CH, _CH)],
                                acc_sh.at[idx_x.at[pl.ds(k * _CH, _CH)]],
                                add=True)

        plsc.subcore_barrier()
        pltpu.sync_copy(acc_sh.at[pl.ds(rbase, rps)],
                        out_hbm.at[cid, pl.ds(rbase, rps)])

    return scatter_kernel(vals, idx1d, zeros)


def _combine_body(p_ref, o_ref):
    o_ref[...] = p_ref[0] + p_ref[1]


def _combine(partials):
    n_acc = partials.shape[1]
    return pl.pallas_call(
        _combine_body,
        out_shape=jax.ShapeDtypeStruct((n_acc, 8), jnp.float32),
    )(partials)


def kernel(force_features, edge_vectors, edge_index_dst, pos,
           W0, b0, W1, b1, W2, b2, W3, b3, W4, b4):
    e, h = force_features.shape
    n = pos.shape[0]

    block_e = 8192
    grid = -(-e // block_e)
    n_acc = -(-n // 128) * 128

    ws = [(0.5 * W0).astype(jnp.bfloat16), (0.5 * W1).astype(jnp.bfloat16),
          (0.5 * W2).astype(jnp.bfloat16), (0.5 * W3).astype(jnp.bfloat16),
          jnp.broadcast_to(W4, (h, 8))]
    bs = [0.5 * b0.reshape(1, h), 0.5 * b1.reshape(1, h),
          0.5 * b2.reshape(1, h), 0.5 * b3.reshape(1, h),
          jnp.broadcast_to(b4.reshape(1, 1), (1, 8))]
    zeros = jnp.zeros((n_acc, 8), jnp.float32)

    vals = _mlp_forces(force_features, edge_vectors.T, ws, bs, block_e, grid)
    partials = _scatter_partials(vals, edge_index_dst, zeros)
    forces_full = _combine(partials)
    return forces_full[:n, :3]
